# Initial kernel scaffold; baseline (speedup 1.0000x reference)
import jax
import jax.numpy as jnp
from jax.experimental import pallas as pl


def kernel(x, edge_index, edge_weight, tc1_w1, tc1_b1, tc1_w2, tc1_b2, tc1_w3, tc1_b3, cheb_W, cheb_b, tc2_w1, tc2_b1, tc2_w2, tc2_b2, tc2_w3, tc2_b3, bn_gamma, bn_beta, lin_w, lin_b):
    raise NotImplementedError("write your pallas kernel here")


# speedup vs baseline: 11.3819x; 11.3819x over previous
import functools

import jax
import jax.numpy as jnp
from jax import lax
from jax.experimental import pallas as pl
from jax.experimental.pallas import tpu as pltpu
from jax.experimental.pallas import tpu_sc as plsc

N_NODES = 10000
F_IN = 128
HID = 64
KT = 3


def _tconv_body(x_ref, w_ref, b_ref, o_ref):
    T, BN, Cin = x_ref.shape
    Tout = T - 2
    C3 = w_ref.shape[1]
    Cout = C3 // 3
    xs = [x_ref[k:k + Tout].reshape(Tout * BN, Cin) for k in range(KT)]
    xcat = jnp.concatenate(xs, axis=1)
    acc = jnp.dot(xcat, w_ref[...], preferred_element_type=jnp.float32)
    acc = acc + b_ref[...]
    P = acc[:, :Cout]
    Q = acc[:, Cout:2 * Cout]
    R = acc[:, 2 * Cout:]
    H = jnp.maximum(P * jax.nn.sigmoid(Q) + R, 0.0)
    o_ref[...] = H.reshape(Tout, BN, Cout)


def _temporal_conv(x, wcat, bcat, bn=1000):
    T, N, Cin = x.shape
    C3 = wcat.shape[1]
    Cout = C3 // 3
    grid = N // bn
    return pl.pallas_call(
        _tconv_body,
        grid=(grid,),
        in_specs=[
            pl.BlockSpec((T, bn, Cin), lambda i: (0, i, 0)),
            pl.BlockSpec((KT * Cin, C3), lambda i: (0, 0)),
            pl.BlockSpec((1, C3), lambda i: (0, 0)),
        ],
        out_specs=pl.BlockSpec((T - 2, bn, Cout), lambda i: (0, i, 0)),
        out_shape=jax.ShapeDtypeStruct((T - 2, N, Cout), jnp.float32),
    )(x, wcat, bcat)


def _cheb_body(t0_ref, y1_ref, y2_ref, w_ref, b_ref, o_ref):
    t0 = t0_ref[...]
    a = jnp.dot(t0, w_ref[0], preferred_element_type=jnp.float32)
    a = a + jnp.dot(y1_ref[...], w_ref[1], preferred_element_type=jnp.float32)
    a = a + jnp.dot(2.0 * y2_ref[...] - t0, w_ref[2],
                    preferred_element_type=jnp.float32)
    o_ref[...] = jnp.maximum(a + b_ref[...], 0.0)


def _cheb_combine(t0f, y1, y2, cheb_W, cheb_b, br=2000):
    R, C = t0f.shape
    grid = R // br
    return pl.pallas_call(
        _cheb_body,
        grid=(grid,),
        in_specs=[
            pl.BlockSpec((br, C), lambda i: (i, 0)),
            pl.BlockSpec((br, C), lambda i: (i, 0)),
            pl.BlockSpec((br, C), lambda i: (i, 0)),
            pl.BlockSpec((3, C, C), lambda i: (0, 0, 0)),
            pl.BlockSpec((1, C), lambda i: (0, 0)),
        ],
        out_specs=pl.BlockSpec((br, C), lambda i: (i, 0)),
        out_shape=jax.ShapeDtypeStruct((R, C), jnp.float32),
    )(t0f, y1, y2, cheb_W, cheb_b.reshape(1, C))


def _pair_body(p_ref, o_ref):
    o_ref[...] = p_ref[0] + p_ref[1]


def _pair_add(p, br=2000):
    _, R, C = p.shape
    grid = R // br
    return pl.pallas_call(
        _pair_body,
        grid=(grid,),
        in_specs=[pl.BlockSpec((2, br, C), lambda i: (0, i, 0))],
        out_specs=pl.BlockSpec((br, C), lambda i: (i, 0)),
        out_shape=jax.ShapeDtypeStruct((R, C), jnp.float32),
    )(p)


def _bn_body(t_ref, g_ref, b_ref, o_ref):
    tb = t_ref[...]
    m = jnp.mean(tb, axis=(0, 2))
    ctr = tb - m[None, :, None]
    v = jnp.mean(ctr * ctr, axis=(0, 2))
    inv = 1.0 / jnp.sqrt(v + 1e-5)
    g = g_ref[0, 0]
    be = b_ref[0, 0]
    scale = (g * inv)[None, :, None]
    shift = (be - m * g * inv)[None, :, None]
    o_ref[...] = jnp.maximum(tb * scale + shift, 0.0)


def _bn_relu(t, gamma, beta, bn=1000):
    M2, N, C = t.shape
    grid = N // bn
    g2 = jnp.broadcast_to(gamma.reshape(grid, 1, bn), (grid, 8, bn))
    b2 = jnp.broadcast_to(beta.reshape(grid, 1, bn), (grid, 8, bn))
    return pl.pallas_call(
        _bn_body,
        grid=(grid,),
        in_specs=[
            pl.BlockSpec((M2, bn, C), lambda i: (0, i, 0)),
            pl.BlockSpec((1, 8, bn), lambda i: (i, 0, 0)),
            pl.BlockSpec((1, 8, bn), lambda i: (i, 0, 0)),
        ],
        out_specs=pl.BlockSpec((M2, bn, C), lambda i: (0, i, 0)),
        out_shape=jax.ShapeDtypeStruct((M2, N, C), jnp.float32),
    )(t, g2, b2)


def _lin_body(h_ref, w_ref, b_ref, o_ref):
    y = jnp.sum(h_ref[...] * w_ref[...], axis=1) + b_ref[0, 0]
    o_ref[...] = y[:, None]


def _final_linear(hf, lin_w, lin_b, br=4000):
    R, C = hf.shape
    grid = R // br
    return pl.pallas_call(
        _lin_body,
        grid=(grid,),
        in_specs=[
            pl.BlockSpec((br, C), lambda i: (i, 0)),
            pl.BlockSpec((1, C), lambda i: (0, 0)),
            pl.BlockSpec((1, 1), lambda i: (0, 0)),
        ],
        out_specs=pl.BlockSpec((br, 1), lambda i: (i, 0)),
        out_shape=jax.ShapeDtypeStruct((R, 1), jnp.float32),
    )(hf, lin_w, lin_b.reshape(1, 1))


_NC, _NS = 2, 16
_NW = _NC * _NS
_EB = 128
_NPAD = 10240
_NPT = _NPAD // _NS


def _sc_mesh():
    return plsc.VectorSubcoreMesh(core_axis_name="c", subcore_axis_name="s")


def _deg_kernel():
    def body(src_hbm, ew_hbm, out_hbm, srcv, ewv, stage, zb, acc):
        c = lax.axis_index("c")
        s = lax.axis_index("s")
        wid = c * _NS + s
        nb = srcv.shape[0]
        pltpu.sync_copy(src_hbm.at[wid], srcv)
        pltpu.sync_copy(ew_hbm.at[wid], ewv)
        z16 = jnp.zeros((16,), jnp.float32)

        def zzb(i, _):
            for j in range(8):
                zb[i, pl.ds(j * 16, 16)] = z16
            return 0

        lax.fori_loop(0, 128, zzb, 0)

        def zacc(i, _):
            pltpu.sync_copy(zb, acc.at[pl.ds(s * _NPT + i * 128, 128)])
            return 0

        lax.fori_loop(0, 5, zacc, 0)
        plsc.subcore_barrier()

        def blk(b, _):
            def put(e, _):
                wv = plsc.load_gather(
                    ewv, [jnp.full((16,), b, jnp.int32),
                          jnp.full((16,), e, jnp.int32)])
                for j in range(8):
                    stage[e, pl.ds(j * 16, 16)] = wv
                return 0

            lax.fori_loop(0, _EB, put, 0)
            pltpu.sync_copy(stage, acc.at[srcv.at[b]], add=True)
            return 0

        lax.fori_loop(0, nb, blk, 0)
        plsc.subcore_barrier()
        pltpu.sync_copy(acc.at[pl.ds(s * _NPT, _NPT)],
                        out_hbm.at[c, pl.ds(s * _NPT, _NPT)])

    return body


def _sc_degree(src3, ew3):
    nb = src3.shape[1]
    return pl.kernel(
        _deg_kernel(),
        out_type=jax.ShapeDtypeStruct((_NC, _NPAD, 128), jnp.float32),
        mesh=_sc_mesh(),
        compiler_params=pltpu.CompilerParams(needs_layout_passes=False),
        scratch_types=[
            pltpu.VMEM((nb, _EB), jnp.int32),
            pltpu.VMEM((nb, _EB), jnp.float32),
            pltpu.VMEM((_EB, 128), jnp.float32),
            pltpu.VMEM((128, 128), jnp.float32),
            pltpu.VMEM_SHARED((_NPAD, 128), jnp.float32),
        ],
    )(src3, ew3)


def _dis_body(p_ref, o_ref):
    deg = jnp.sum(p_ref[0] + p_ref[1], axis=1) * (1.0 / 128.0)
    safe = jnp.where(deg > 0, deg, 1.0)
    o_ref[...] = jnp.where(deg > 0, lax.rsqrt(safe), 0.0)[None, :]


def _dis_from_partials(part):
    out = pl.pallas_call(
        _dis_body,
        out_shape=jax.ShapeDtypeStruct((1, _NPAD), jnp.float32),
    )(part)
    return out.reshape(_NPAD)


def _lw_kernel():
    def body(src_hbm, dst_hbm, ew_hbm, dis_hbm, lw_hbm,
             srcv, dstv, ewv, disv, lwv):
        c = lax.axis_index("c")
        s = lax.axis_index("s")
        wid = c * _NS + s
        nb = srcv.shape[0]
        pltpu.sync_copy(src_hbm.at[wid], srcv)
        pltpu.sync_copy(dst_hbm.at[wid], dstv)
        pltpu.sync_copy(ew_hbm.at[wid], ewv)
        pltpu.sync_copy(dis_hbm, disv)

        def blk(b, _):
            def j16(j, _):
                sl = pl.ds(j * 16, 16)
                s16 = srcv[b, sl]
                d16 = dstv[b, sl]
                e16 = ewv[b, sl]
                dsc = plsc.load_gather(disv, [s16])
                ddc = plsc.load_gather(disv, [d16])
                lwv[b, sl] = (0.0 - dsc) * e16 * ddc
                return 0

            lax.fori_loop(0, _EB // 16, j16, 0)
            return 0

        lax.fori_loop(0, nb, blk, 0)
        pltpu.sync_copy(lwv, lw_hbm.at[wid])

    return body


def _sc_edge_norm(src3, dst3, ew3, dis):
    nb = src3.shape[1]
    return pl.kernel(
        _lw_kernel(),
        out_type=jax.ShapeDtypeStruct((_NW, nb, _EB), jnp.float32),
        mesh=_sc_mesh(),
        compiler_params=pltpu.CompilerParams(needs_layout_passes=False),
        scratch_types=[
            pltpu.VMEM((nb, _EB), jnp.int32),
            pltpu.VMEM((nb, _EB), jnp.int32),
            pltpu.VMEM((nb, _EB), jnp.float32),
            pltpu.VMEM((_NPAD,), jnp.float32),
            pltpu.VMEM((nb, _EB), jnp.float32),
        ],
    )(src3, dst3, ew3, dis)


_PW = 128


def _lhat_body(xf_hbm, srcall_hbm, dst_hbm, lw_hbm, out_hbm,
               dstv, lwv, srcmv, rows, zb, acc, sem):
    c = lax.axis_index("c")
    s = lax.axis_index("s")
    wid = c * _NS + s
    nb = dstv.shape[0]
    M = srcall_hbm.shape[0]
    pltpu.sync_copy(dst_hbm.at[wid], dstv)
    pltpu.sync_copy(lw_hbm.at[wid], lwv)
    z16 = jnp.zeros((16,), jnp.float32)

    def zzb(i, _):
        for j in range(_PW // 16):
            zb[i, pl.ds(j * 16, 16)] = z16
        return 0

    lax.fori_loop(0, 128, zzb, 0)

    def slice_loop(m, _):
        pltpu.sync_copy(srcall_hbm.at[m, wid], srcmv)

        def zacc(i, _):
            pltpu.sync_copy(zb, acc.at[pl.ds(s * _NPT + i * 128, 128)])
            return 0

        lax.fori_loop(0, 5, zacc, 0)
        plsc.subcore_barrier()

        def blk(b, _):
            pltpu.async_copy(xf_hbm.at[srcmv.at[b]], rows, sem).wait()

            def scale(e, _):
                wv = plsc.load_gather(
                    lwv, [jnp.full((16,), b, jnp.int32),
                          jnp.full((16,), e, jnp.int32)])
                for j in range(_PW // 16):
                    sl = pl.ds(j * 16, 16)
                    rows[e, sl] = rows[e, sl] * wv
                return 0

            lax.fori_loop(0, _EB, scale, 0)
            pltpu.sync_copy(rows, acc.at[dstv.at[b]], add=True)
            return 0

        lax.fori_loop(0, nb, blk, 0)
        plsc.subcore_barrier()
        pltpu.sync_copy(acc.at[pl.ds(s * _NPT, _NPT)],
                        out_hbm.at[c, m, pl.ds(s * _NPT, _NPT)])
        plsc.subcore_barrier()
        return 0

    lax.fori_loop(0, M, slice_loop, 0)


@functools.lru_cache(maxsize=None)
def _lhat_call(Mp, nb):
    return pl.kernel(
        _lhat_body,
        out_type=jax.ShapeDtypeStruct((_NC, Mp, _NPAD, _PW), jnp.float32),
        mesh=_sc_mesh(),
        compiler_params=pltpu.CompilerParams(needs_layout_passes=False),
        scratch_types=[
            pltpu.VMEM((nb, _EB), jnp.int32),
            pltpu.VMEM((nb, _EB), jnp.float32),
            pltpu.VMEM((nb, _EB), jnp.int32),
            pltpu.VMEM((_EB, _PW), jnp.float32),
            pltpu.VMEM((128, _PW), jnp.float32),
            pltpu.VMEM_SHARED((_NPAD, _PW), jnp.float32),
            pltpu.SemaphoreType.DMA,
        ],
    )


def _pairpad_body(p_ref, o_ref):
    o_ref[...] = (p_ref[0, 0] + p_ref[1, 0])[None]


def _pair_add_padded(p, bn=2000):
    _, Mp, _, C = p.shape
    grid_n = N_NODES // bn
    out = pl.pallas_call(
        _pairpad_body,
        grid=(Mp, grid_n),
        in_specs=[pl.BlockSpec((2, 1, bn, C), lambda m, i: (0, m, i, 0))],
        out_specs=pl.BlockSpec((1, bn, C), lambda m, i: (m, i, 0)),
        out_shape=jax.ShapeDtypeStruct((Mp, N_NODES, C), jnp.float32),
    )(p)
    return out.reshape(Mp * N_NODES, C)


def _sc_lhat(xf, src_all, dst3, lw3, Mp):
    nb = dst3.shape[1]
    part = _lhat_call(Mp, nb)(xf, src_all[:Mp], dst3, lw3)
    return _pair_add_padded(part)


def _prep_tconv_weights(w1, b1, w2, b2, w3, b3):
    def per_branch(w):
        m = jnp.transpose(w[:, :, 0, :], (2, 1, 0))
        return m.reshape(-1, m.shape[2])

    Wcat = jnp.concatenate([per_branch(w1), per_branch(w2), per_branch(w3)], axis=1)
    bcat = jnp.concatenate([b1, b2, b3]).reshape(1, -1)
    return Wcat, bcat


def kernel(x, edge_index, edge_weight,
           tc1_w1, tc1_b1, tc1_w2, tc1_b2, tc1_w3, tc1_b3,
           cheb_W, cheb_b,
           tc2_w1, tc2_b1, tc2_w2, tc2_b2, tc2_w3, tc2_b3,
           bn_gamma, bn_beta, lin_w, lin_b):
    src = edge_index[0].astype(jnp.int32)
    dst = edge_index[1].astype(jnp.int32)
    W1cat, b1cat = _prep_tconv_weights(tc1_w1, tc1_b1, tc1_w2, tc1_b2, tc1_w3, tc1_b3)
    W2cat, b2cat = _prep_tconv_weights(tc2_w1, tc2_b1, tc2_w2, tc2_b2, tc2_w3, tc2_b3)

    E = src.shape[0]
    nb = -(-E // (_NW * _EB))
    EP = _NW * _EB * nb
    pad = EP - E
    srcp = jnp.concatenate([src, jnp.zeros((pad,), jnp.int32)])
    dstp = jnp.concatenate([dst, jnp.zeros((pad,), jnp.int32)])
    ewp = jnp.concatenate([edge_weight, jnp.zeros((pad,), jnp.float32)])
    src3 = srcp.reshape(_NW, nb, _EB)
    dst3 = dstp.reshape(_NW, nb, _EB)
    ew3 = ewp.reshape(_NW, nb, _EB)
    Mpmax = (x.shape[1] - 2) // 2
    src_all = src3[None] + (jnp.arange(Mpmax, dtype=jnp.int32) * N_NODES)[:, None, None, None]

    part = _sc_degree(src3, ew3)
    dis = _dis_from_partials(part)
    lw3 = _sc_edge_norm(src3, dst3, ew3, dis)

    z = jnp.zeros((3, HID, HID), jnp.float32)
    Wbd = jnp.concatenate([
        jnp.concatenate([cheb_W, z], axis=2),
        jnp.concatenate([z, cheb_W], axis=2),
    ], axis=1)
    bbd = jnp.concatenate([cheb_b, cheb_b])

    h = x[0]
    for _ in range(3):
        t0 = _temporal_conv(h, W1cat, b1cat)
        M = t0.shape[0]
        Mp = M // 2
        t0p = (t0.reshape(Mp, 2, N_NODES, HID)
               .transpose(0, 2, 1, 3).reshape(Mp * N_NODES, _PW))
        y1 = _sc_lhat(t0p, src_all, dst3, lw3, Mp)
        y2 = _sc_lhat(y1, src_all, dst3, lw3, Mp)
        g = _cheb_combine(t0p, y1, y2, Wbd, bbd)
        gu = (g.reshape(Mp, N_NODES, 2, HID)
              .transpose(0, 2, 1, 3).reshape(M, N_NODES, HID))
        t2 = _temporal_conv(gu, W2cat, b2cat)
        h = _bn_relu(t2, bn_gamma, bn_beta)
    M2 = h.shape[0]
    out = _final_linear(h.reshape(M2 * N_NODES, F_IN), lin_w, lin_b)
    return out.reshape(1, M2, N_NODES, 1)


# speedup vs baseline: 13.7750x; 1.2103x over previous
import functools

import jax
import jax.numpy as jnp
from jax import lax
from jax.experimental import pallas as pl
from jax.experimental.pallas import tpu as pltpu
from jax.experimental.pallas import tpu_sc as plsc

N_NODES = 10000
F_IN = 128
HID = 64
KT = 3


def _tconv_body(x_ref, w_ref, b_ref, o_ref):
    T, BN, Cin = x_ref.shape
    Tout = T - 2
    C3 = w_ref.shape[1]
    Cout = C3 // 3
    xs = [x_ref[k:k + Tout].reshape(Tout * BN, Cin) for k in range(KT)]
    xcat = jnp.concatenate(xs, axis=1)
    acc = jnp.dot(xcat, w_ref[...], preferred_element_type=jnp.float32)
    acc = acc + b_ref[...]
    P = acc[:, :Cout]
    Q = acc[:, Cout:2 * Cout]
    R = acc[:, 2 * Cout:]
    H = jnp.maximum(P * jax.nn.sigmoid(Q) + R, 0.0)
    o_ref[...] = H.reshape(Tout, BN, Cout)


def _temporal_conv(x, wcat, bcat, bn=1000):
    T, N, Cin = x.shape
    C3 = wcat.shape[1]
    Cout = C3 // 3
    grid = N // bn
    return pl.pallas_call(
        _tconv_body,
        grid=(grid,),
        in_specs=[
            pl.BlockSpec((T, bn, Cin), lambda i: (0, i, 0)),
            pl.BlockSpec((KT * Cin, C3), lambda i: (0, 0)),
            pl.BlockSpec((1, C3), lambda i: (0, 0)),
        ],
        out_specs=pl.BlockSpec((T - 2, bn, Cout), lambda i: (0, i, 0)),
        out_shape=jax.ShapeDtypeStruct((T - 2, N, Cout), jnp.float32),
    )(x, wcat, bcat)


def _cheb_body(t0_ref, y1_ref, y2_ref, w_ref, b_ref, o_ref):
    t0 = t0_ref[...]
    a = jnp.dot(t0, w_ref[0], preferred_element_type=jnp.float32)
    a = a + jnp.dot(y1_ref[...], w_ref[1], preferred_element_type=jnp.float32)
    a = a + jnp.dot(2.0 * y2_ref[...] - t0, w_ref[2],
                    preferred_element_type=jnp.float32)
    o_ref[...] = jnp.maximum(a + b_ref[...], 0.0)


def _cheb_combine(t0f, y1, y2, cheb_W, cheb_b, br=2000):
    R, C = t0f.shape
    grid = R // br
    return pl.pallas_call(
        _cheb_body,
        grid=(grid,),
        in_specs=[
            pl.BlockSpec((br, C), lambda i: (i, 0)),
            pl.BlockSpec((br, C), lambda i: (i, 0)),
            pl.BlockSpec((br, C), lambda i: (i, 0)),
            pl.BlockSpec((3, C, C), lambda i: (0, 0, 0)),
            pl.BlockSpec((1, C), lambda i: (0, 0)),
        ],
        out_specs=pl.BlockSpec((br, C), lambda i: (i, 0)),
        out_shape=jax.ShapeDtypeStruct((R, C), jnp.float32),
    )(t0f, y1, y2, cheb_W, cheb_b.reshape(1, C))


def _pair_body(p_ref, o_ref):
    o_ref[...] = p_ref[0] + p_ref[1]


def _pair_add(p, br=2000):
    _, R, C = p.shape
    grid = R // br
    return pl.pallas_call(
        _pair_body,
        grid=(grid,),
        in_specs=[pl.BlockSpec((2, br, C), lambda i: (0, i, 0))],
        out_specs=pl.BlockSpec((br, C), lambda i: (i, 0)),
        out_shape=jax.ShapeDtypeStruct((R, C), jnp.float32),
    )(p)


def _bn_body(t_ref, g_ref, b_ref, o_ref):
    tb = t_ref[...]
    m = jnp.mean(tb, axis=(0, 2))
    ctr = tb - m[None, :, None]
    v = jnp.mean(ctr * ctr, axis=(0, 2))
    inv = 1.0 / jnp.sqrt(v + 1e-5)
    g = g_ref[0, 0]
    be = b_ref[0, 0]
    scale = (g * inv)[None, :, None]
    shift = (be - m * g * inv)[None, :, None]
    o_ref[...] = jnp.maximum(tb * scale + shift, 0.0)


def _bn_relu(t, gamma, beta, bn=1000):
    M2, N, C = t.shape
    grid = N // bn
    g2 = jnp.broadcast_to(gamma.reshape(grid, 1, bn), (grid, 8, bn))
    b2 = jnp.broadcast_to(beta.reshape(grid, 1, bn), (grid, 8, bn))
    return pl.pallas_call(
        _bn_body,
        grid=(grid,),
        in_specs=[
            pl.BlockSpec((M2, bn, C), lambda i: (0, i, 0)),
            pl.BlockSpec((1, 8, bn), lambda i: (i, 0, 0)),
            pl.BlockSpec((1, 8, bn), lambda i: (i, 0, 0)),
        ],
        out_specs=pl.BlockSpec((M2, bn, C), lambda i: (0, i, 0)),
        out_shape=jax.ShapeDtypeStruct((M2, N, C), jnp.float32),
    )(t, g2, b2)


def _lin_body(h_ref, w_ref, b_ref, o_ref):
    y = jnp.sum(h_ref[...] * w_ref[...], axis=1) + b_ref[0, 0]
    o_ref[...] = y[:, None]


def _final_linear(hf, lin_w, lin_b, br=4000):
    R, C = hf.shape
    grid = R // br
    return pl.pallas_call(
        _lin_body,
        grid=(grid,),
        in_specs=[
            pl.BlockSpec((br, C), lambda i: (i, 0)),
            pl.BlockSpec((1, C), lambda i: (0, 0)),
            pl.BlockSpec((1, 1), lambda i: (0, 0)),
        ],
        out_specs=pl.BlockSpec((br, 1), lambda i: (i, 0)),
        out_shape=jax.ShapeDtypeStruct((R, 1), jnp.float32),
    )(hf, lin_w, lin_b.reshape(1, 1))


_NC, _NS = 2, 16
_NW = _NC * _NS
_EB = 128
_NPAD = 10240
_NPT = _NPAD // _NS


def _sc_mesh():
    return plsc.VectorSubcoreMesh(core_axis_name="c", subcore_axis_name="s")


def _dis_body(p_ref, o_ref):
    deg = jnp.sum(p_ref[0] + p_ref[1], axis=1) * (1.0 / 128.0)
    safe = jnp.where(deg > 0, deg, 1.0)
    o_ref[...] = jnp.where(deg > 0, lax.rsqrt(safe), 0.0)[None, :]


def _dis_from_partials(part):
    out = pl.pallas_call(
        _dis_body,
        out_shape=jax.ShapeDtypeStruct((1, _NPAD), jnp.float32),
    )(part)
    return out.reshape(_NPAD)


def _lw_kernel():
    def body(src_hbm, dst_hbm, ew_hbm, dis_hbm, lw_hbm,
             srcv, dstv, ewv, disv, lwv):
        c = lax.axis_index("c")
        s = lax.axis_index("s")
        wid = c * _NS + s
        nb = srcv.shape[0]
        pltpu.sync_copy(src_hbm.at[wid], srcv)
        pltpu.sync_copy(dst_hbm.at[wid], dstv)
        pltpu.sync_copy(ew_hbm.at[wid], ewv)
        pltpu.sync_copy(dis_hbm, disv)

        def blk(b, _):
            def j16(j, _):
                sl = pl.ds(j * 16, 16)
                s16 = srcv[b, sl]
                d16 = dstv[b, sl]
                e16 = ewv[b, sl]
                dsc = plsc.load_gather(disv, [s16])
                ddc = plsc.load_gather(disv, [d16])
                lwv[b, sl] = (0.0 - dsc) * e16 * ddc
                return 0

            lax.fori_loop(0, _EB // 16, j16, 0)
            return 0

        lax.fori_loop(0, nb, blk, 0)
        pltpu.sync_copy(lwv, lw_hbm.at[wid])

    return body


def _sc_edge_norm(src3, dst3, ew3, dis):
    nb = src3.shape[1]
    return pl.kernel(
        _lw_kernel(),
        out_type=jax.ShapeDtypeStruct((_NW, nb, _EB), jnp.float32),
        mesh=_sc_mesh(),
        compiler_params=pltpu.CompilerParams(needs_layout_passes=False),
        scratch_types=[
            pltpu.VMEM((nb, _EB), jnp.int32),
            pltpu.VMEM((nb, _EB), jnp.int32),
            pltpu.VMEM((nb, _EB), jnp.float32),
            pltpu.VMEM((_NPAD,), jnp.float32),
            pltpu.VMEM((nb, _EB), jnp.float32),
        ],
    )(src3, dst3, ew3, dis)


_PW = 128


def _lhat_body(xf_hbm, srcall_hbm, dst_hbm, lw_hbm, out_hbm,
               dstv, lwv, srcmv, rows0, rows1, acc, gs0, gs1):
    c = lax.axis_index("c")
    s = lax.axis_index("s")
    wid = c * _NS + s
    nb = dstv.shape[0]
    M = srcall_hbm.shape[0]
    pltpu.sync_copy(dst_hbm.at[wid], dstv)
    pltpu.sync_copy(lw_hbm.at[wid], lwv)
    z16 = jnp.zeros((16,), jnp.float32)

    bufs = ((rows0, gs0), (rows1, gs1))

    def g_start(b, k):
        pltpu.async_copy(xf_hbm.at[srcmv.at[b]], bufs[k][0], bufs[k][1])

    def g_wait(b, k):
        pltpu.make_async_copy(xf_hbm.at[srcmv.at[b]], bufs[k][0],
                              bufs[k][1]).wait()

    def do_scale(b, k):
        rbuf = bufs[k][0]

        def scale(e, _):
            wv = plsc.load_gather(
                lwv, [jnp.full((16,), b, jnp.int32),
                      jnp.full((16,), e, jnp.int32)])
            for j in range(_PW // 16):
                sl = pl.ds(j * 16, 16)
                rbuf[e, sl] = rbuf[e, sl] * wv
            return 0

        lax.fori_loop(0, _EB, scale, 0)

    def s_sync(b, k):
        pltpu.sync_copy(bufs[k][0], acc.at[dstv.at[b]], add=True)

    def slice_loop(m, _):
        pltpu.sync_copy(srcall_hbm.at[m, wid], srcmv)

        def zrows(i, _):
            for j in range(_PW // 16):
                rows0[i, pl.ds(j * 16, 16)] = z16
            return 0

        lax.fori_loop(0, _EB, zrows, 0)

        def zacc(i, _):
            pltpu.sync_copy(rows0, acc.at[pl.ds(s * _NPT + i * 128, 128)])
            return 0

        lax.fori_loop(0, 5, zacc, 0)
        plsc.subcore_barrier()

        g_start(0, 0)
        g_start(1, 1)

        def outer(b2, _):
            b = 2 * b2
            g_wait(b, 0)
            do_scale(b, 0)
            s_sync(b, 0)
            g_start(b + 2, 0)
            g_wait(b + 1, 1)
            do_scale(b + 1, 1)
            s_sync(b + 1, 1)
            g_start(b + 3, 1)
            return 0

        lax.fori_loop(0, (nb - 2) // 2, outer, 0)
        b = nb - 2
        g_wait(b, 0)
        do_scale(b, 0)
        s_sync(b, 0)
        g_wait(b + 1, 1)
        do_scale(b + 1, 1)
        s_sync(b + 1, 1)
        plsc.subcore_barrier()
        pltpu.sync_copy(acc.at[pl.ds(s * _NPT, _NPT)],
                        out_hbm.at[c, m, pl.ds(s * _NPT, _NPT)])
        plsc.subcore_barrier()
        return 0

    lax.fori_loop(0, M, slice_loop, 0)


@functools.lru_cache(maxsize=None)
def _lhat_call(Mp, nb):
    return pl.kernel(
        _lhat_body,
        out_type=jax.ShapeDtypeStruct((_NC, Mp, _NPAD, _PW), jnp.float32),
        mesh=_sc_mesh(),
        name=f"lhat_m{Mp}",
        compiler_params=pltpu.CompilerParams(needs_layout_passes=False),
        scratch_types=[
            pltpu.VMEM((nb, _EB), jnp.int32),
            pltpu.VMEM((nb, _EB), jnp.float32),
            pltpu.VMEM((nb, _EB), jnp.int32),
            pltpu.VMEM((_EB, _PW), jnp.float32),
            pltpu.VMEM((_EB, _PW), jnp.float32),
            pltpu.VMEM_SHARED((_NPAD, _PW), jnp.float32),
            pltpu.SemaphoreType.DMA,
            pltpu.SemaphoreType.DMA,
        ],
    )


def _pairpad_body(p_ref, o_ref):
    o_ref[...] = (p_ref[0, 0] + p_ref[1, 0])[None]


def _pair_add_padded(p, bn=2000):
    _, Mp, _, C = p.shape
    grid_n = N_NODES // bn
    out = pl.pallas_call(
        _pairpad_body,
        grid=(Mp, grid_n),
        in_specs=[pl.BlockSpec((2, 1, bn, C), lambda m, i: (0, m, i, 0))],
        out_specs=pl.BlockSpec((1, bn, C), lambda m, i: (m, i, 0)),
        out_shape=jax.ShapeDtypeStruct((Mp, N_NODES, C), jnp.float32),
    )(p)
    return out.reshape(Mp * N_NODES, C)


def _sc_lhat(xf, src_all, dst3, lw3, Mp):
    nb = dst3.shape[1]
    part = _lhat_call(Mp, nb)(xf, src_all[:Mp], dst3, lw3)
    return _pair_add_padded(part)


def _prep_tconv_weights(w1, b1, w2, b2, w3, b3):
    def per_branch(w):
        m = jnp.transpose(w[:, :, 0, :], (2, 1, 0))
        return m.reshape(-1, m.shape[2])

    Wcat = jnp.concatenate([per_branch(w1), per_branch(w2), per_branch(w3)], axis=1)
    bcat = jnp.concatenate([b1, b2, b3]).reshape(1, -1)
    return Wcat, bcat


def kernel(x, edge_index, edge_weight,
           tc1_w1, tc1_b1, tc1_w2, tc1_b2, tc1_w3, tc1_b3,
           cheb_W, cheb_b,
           tc2_w1, tc2_b1, tc2_w2, tc2_b2, tc2_w3, tc2_b3,
           bn_gamma, bn_beta, lin_w, lin_b):
    src = edge_index[0].astype(jnp.int32)
    dst = edge_index[1].astype(jnp.int32)
    W1cat, b1cat = _prep_tconv_weights(tc1_w1, tc1_b1, tc1_w2, tc1_b2, tc1_w3, tc1_b3)
    W2cat, b2cat = _prep_tconv_weights(tc2_w1, tc2_b1, tc2_w2, tc2_b2, tc2_w3, tc2_b3)

    E = src.shape[0]
    nb = -(-E // (_NW * _EB))
    EP = _NW * _EB * nb
    pad = EP - E
    srcp = jnp.concatenate([src, jnp.zeros((pad,), jnp.int32)])
    dstp = jnp.concatenate([dst, jnp.zeros((pad,), jnp.int32)])
    ewp = jnp.concatenate([edge_weight, jnp.zeros((pad,), jnp.float32)])
    src3 = srcp.reshape(_NW, nb, _EB)
    dst3 = dstp.reshape(_NW, nb, _EB)
    ew3 = ewp.reshape(_NW, nb, _EB)
    Mpmax = (x.shape[1] - 2) // 2
    src_all = src3[None] + (jnp.arange(Mpmax, dtype=jnp.int32) * N_NODES)[:, None, None, None]

    ones_tab = jnp.ones((2 * N_NODES, _PW), jnp.float32)
    deg_srcall = jnp.stack([src3, src3 + N_NODES])
    part = _lhat_call(2, nb)(ones_tab, deg_srcall, src3, ew3)
    dis = _dis_from_partials(part[:, 0])
    lw3 = _sc_edge_norm(src3, dst3, ew3, dis)

    z = jnp.zeros((3, HID, HID), jnp.float32)
    Wbd = jnp.concatenate([
        jnp.concatenate([cheb_W, z], axis=2),
        jnp.concatenate([z, cheb_W], axis=2),
    ], axis=1)
    bbd = jnp.concatenate([cheb_b, cheb_b])

    h = x[0]
    for _ in range(3):
        t0 = _temporal_conv(h, W1cat, b1cat)
        M = t0.shape[0]
        Mp = M // 2
        t0p = (t0.reshape(Mp, 2, N_NODES, HID)
               .transpose(0, 2, 1, 3).reshape(Mp * N_NODES, _PW))
        y1 = _sc_lhat(t0p, src_all, dst3, lw3, Mp)
        y2 = _sc_lhat(y1, src_all, dst3, lw3, Mp)
        g = _cheb_combine(t0p, y1, y2, Wbd, bbd)
        gu = (g.reshape(Mp, N_NODES, 2, HID)
              .transpose(0, 2, 1, 3).reshape(M, N_NODES, HID))
        t2 = _temporal_conv(gu, W2cat, b2cat)
        h = _bn_relu(t2, bn_gamma, bn_beta)
    M2 = h.shape[0]
    out = _final_linear(h.reshape(M2 * N_NODES, F_IN), lin_w, lin_b)
    return out.reshape(1, M2, N_NODES, 1)


# speedup vs baseline: 13.9362x; 1.0117x over previous
import functools

import jax
import jax.numpy as jnp
from jax import lax
from jax.experimental import pallas as pl
from jax.experimental.pallas import tpu as pltpu
from jax.experimental.pallas import tpu_sc as plsc

N_NODES = 10000
F_IN = 128
HID = 64
KT = 3


def _tconv_body(x_ref, w_ref, b_ref, o_ref):
    T, BN, Cin = x_ref.shape
    Tout = T - 2
    C3 = w_ref.shape[1]
    Cout = C3 // 3
    xs = [x_ref[k:k + Tout].reshape(Tout * BN, Cin) for k in range(KT)]
    xcat = jnp.concatenate(xs, axis=1)
    acc = jnp.dot(xcat, w_ref[...], preferred_element_type=jnp.float32)
    acc = acc + b_ref[...]
    P = acc[:, :Cout]
    Q = acc[:, Cout:2 * Cout]
    R = acc[:, 2 * Cout:]
    H = jnp.maximum(P * jax.nn.sigmoid(Q) + R, 0.0)
    o_ref[...] = H.reshape(Tout, BN, Cout)


def _temporal_conv(x, wcat, bcat, bn=1000):
    T, N, Cin = x.shape
    C3 = wcat.shape[1]
    Cout = C3 // 3
    grid = N // bn
    return pl.pallas_call(
        _tconv_body,
        grid=(grid,),
        in_specs=[
            pl.BlockSpec((T, bn, Cin), lambda i: (0, i, 0)),
            pl.BlockSpec((KT * Cin, C3), lambda i: (0, 0)),
            pl.BlockSpec((1, C3), lambda i: (0, 0)),
        ],
        out_specs=pl.BlockSpec((T - 2, bn, Cout), lambda i: (0, i, 0)),
        out_shape=jax.ShapeDtypeStruct((T - 2, N, Cout), jnp.float32),
    )(x, wcat, bcat)


def _cheb_body(t0_ref, y1_ref, y2_ref, w_ref, b_ref, o_ref):
    t0 = t0_ref[...]
    a = jnp.dot(t0, w_ref[0], preferred_element_type=jnp.float32)
    a = a + jnp.dot(y1_ref[...], w_ref[1], preferred_element_type=jnp.float32)
    a = a + jnp.dot(2.0 * y2_ref[...] - t0, w_ref[2],
                    preferred_element_type=jnp.float32)
    o_ref[...] = jnp.maximum(a + b_ref[...], 0.0)


def _cheb_combine(t0f, y1, y2, cheb_W, cheb_b, br=2000):
    R, C = t0f.shape
    grid = R // br
    return pl.pallas_call(
        _cheb_body,
        grid=(grid,),
        in_specs=[
            pl.BlockSpec((br, C), lambda i: (i, 0)),
            pl.BlockSpec((br, C), lambda i: (i, 0)),
            pl.BlockSpec((br, C), lambda i: (i, 0)),
            pl.BlockSpec((3, C, C), lambda i: (0, 0, 0)),
            pl.BlockSpec((1, C), lambda i: (0, 0)),
        ],
        out_specs=pl.BlockSpec((br, C), lambda i: (i, 0)),
        out_shape=jax.ShapeDtypeStruct((R, C), jnp.float32),
    )(t0f, y1, y2, cheb_W, cheb_b.reshape(1, C))


def _pair_body(p_ref, o_ref):
    o_ref[...] = p_ref[0] + p_ref[1]


def _pair_add(p, br=2000):
    _, R, C = p.shape
    grid = R // br
    return pl.pallas_call(
        _pair_body,
        grid=(grid,),
        in_specs=[pl.BlockSpec((2, br, C), lambda i: (0, i, 0))],
        out_specs=pl.BlockSpec((br, C), lambda i: (i, 0)),
        out_shape=jax.ShapeDtypeStruct((R, C), jnp.float32),
    )(p)


def _bn_body(t_ref, g_ref, b_ref, o_ref):
    tb = t_ref[...]
    m = jnp.mean(tb, axis=(0, 2))
    ctr = tb - m[None, :, None]
    v = jnp.mean(ctr * ctr, axis=(0, 2))
    inv = 1.0 / jnp.sqrt(v + 1e-5)
    g = g_ref[0, 0]
    be = b_ref[0, 0]
    scale = (g * inv)[None, :, None]
    shift = (be - m * g * inv)[None, :, None]
    o_ref[...] = jnp.maximum(tb * scale + shift, 0.0)


def _bn_relu(t, gamma, beta, bn=1000):
    M2, N, C = t.shape
    grid = N // bn
    g2 = jnp.broadcast_to(gamma.reshape(grid, 1, bn), (grid, 8, bn))
    b2 = jnp.broadcast_to(beta.reshape(grid, 1, bn), (grid, 8, bn))
    return pl.pallas_call(
        _bn_body,
        grid=(grid,),
        in_specs=[
            pl.BlockSpec((M2, bn, C), lambda i: (0, i, 0)),
            pl.BlockSpec((1, 8, bn), lambda i: (i, 0, 0)),
            pl.BlockSpec((1, 8, bn), lambda i: (i, 0, 0)),
        ],
        out_specs=pl.BlockSpec((M2, bn, C), lambda i: (0, i, 0)),
        out_shape=jax.ShapeDtypeStruct((M2, N, C), jnp.float32),
    )(t, g2, b2)


def _lin_body(h_ref, w_ref, b_ref, o_ref):
    y = jnp.sum(h_ref[...] * w_ref[...], axis=1) + b_ref[0, 0]
    o_ref[...] = y[:, None]


def _final_linear(hf, lin_w, lin_b, br=4000):
    R, C = hf.shape
    grid = R // br
    return pl.pallas_call(
        _lin_body,
        grid=(grid,),
        in_specs=[
            pl.BlockSpec((br, C), lambda i: (i, 0)),
            pl.BlockSpec((1, C), lambda i: (0, 0)),
            pl.BlockSpec((1, 1), lambda i: (0, 0)),
        ],
        out_specs=pl.BlockSpec((br, 1), lambda i: (i, 0)),
        out_shape=jax.ShapeDtypeStruct((R, 1), jnp.float32),
    )(hf, lin_w, lin_b.reshape(1, 1))


_NC, _NS = 2, 16
_NW = _NC * _NS
_EB = 128
_NPAD = 10240
_NPT = _NPAD // _NS


def _sc_mesh():
    return plsc.VectorSubcoreMesh(core_axis_name="c", subcore_axis_name="s")


def _dis_body(p_ref, o_ref):
    deg = jnp.sum(p_ref[0] + p_ref[1], axis=1) * (1.0 / 128.0)
    safe = jnp.where(deg > 0, deg, 1.0)
    o_ref[...] = jnp.where(deg > 0, lax.rsqrt(safe), 0.0)[None, :]


def _dis_from_partials(part):
    out = pl.pallas_call(
        _dis_body,
        out_shape=jax.ShapeDtypeStruct((1, _NPAD), jnp.float32),
    )(part)
    return out.reshape(_NPAD)


def _lw_kernel():
    def body(src_hbm, dst_hbm, ew_hbm, dis_hbm, lw_hbm,
             srcv, dstv, ewv, disv, lwv):
        c = lax.axis_index("c")
        s = lax.axis_index("s")
        wid = c * _NS + s
        nb = srcv.shape[0]
        pltpu.sync_copy(src_hbm.at[wid], srcv)
        pltpu.sync_copy(dst_hbm.at[wid], dstv)
        pltpu.sync_copy(ew_hbm.at[wid], ewv)
        pltpu.sync_copy(dis_hbm, disv)

        def blk(b, _):
            def j16(j, _):
                sl = pl.ds(j * 16, 16)
                s16 = srcv[b, sl]
                d16 = dstv[b, sl]
                e16 = ewv[b, sl]
                dsc = plsc.load_gather(disv, [s16])
                ddc = plsc.load_gather(disv, [d16])
                lwv[b, sl] = (0.0 - dsc) * e16 * ddc
                return 0

            lax.fori_loop(0, _EB // 16, j16, 0)
            return 0

        lax.fori_loop(0, nb, blk, 0)
        pltpu.sync_copy(lwv, lw_hbm.at[wid])

    return body


def _sc_edge_norm(src3, dst3, ew3, dis):
    nb = src3.shape[1]
    return pl.kernel(
        _lw_kernel(),
        out_type=jax.ShapeDtypeStruct((_NW, nb, _EB), jnp.float32),
        mesh=_sc_mesh(),
        compiler_params=pltpu.CompilerParams(needs_layout_passes=False),
        scratch_types=[
            pltpu.VMEM((nb, _EB), jnp.int32),
            pltpu.VMEM((nb, _EB), jnp.int32),
            pltpu.VMEM((nb, _EB), jnp.float32),
            pltpu.VMEM((_NPAD,), jnp.float32),
            pltpu.VMEM((nb, _EB), jnp.float32),
        ],
    )(src3, dst3, ew3, dis)


_PW = 128


def _lhat_body(xf_hbm, srcall_hbm, dst_hbm, lw_hbm, out_hbm,
               dstv, lwv, srcmv, rows0, rows1, acc, gs0, gs1):
    c = lax.axis_index("c")
    s = lax.axis_index("s")
    wid = c * _NS + s
    nb = dstv.shape[0]
    M = srcall_hbm.shape[0]
    pltpu.sync_copy(dst_hbm.at[wid], dstv)
    pltpu.sync_copy(lw_hbm.at[wid], lwv)
    z16 = jnp.zeros((16,), jnp.float32)

    bufs = ((rows0, gs0), (rows1, gs1))

    def g_start(b, k):
        pltpu.async_copy(xf_hbm.at[srcmv.at[b]], bufs[k][0], bufs[k][1])

    def g_wait(b, k):
        pltpu.make_async_copy(xf_hbm.at[srcmv.at[b]], bufs[k][0],
                              bufs[k][1]).wait()

    def do_scale(b, k):
        rbuf = bufs[k][0]

        @plsc.parallel_loop(0, _EB, unroll=4)
        def scale(e):
            wv = plsc.load_gather(
                lwv, [jnp.full((16,), b, jnp.int32),
                      jnp.full((16,), e, jnp.int32)])
            for j in range(_PW // 16):
                sl = pl.ds(j * 16, 16)
                rbuf[e, sl] = rbuf[e, sl] * wv

    def s_sync(b, k):
        pltpu.sync_copy(bufs[k][0], acc.at[dstv.at[b]], add=True)

    def slice_loop(m, _):
        pltpu.sync_copy(srcall_hbm.at[m, wid], srcmv)

        def zrows(i, _):
            for j in range(_PW // 16):
                rows0[i, pl.ds(j * 16, 16)] = z16
            return 0

        lax.fori_loop(0, _EB, zrows, 0)

        def zacc(i, _):
            pltpu.sync_copy(rows0, acc.at[pl.ds(s * _NPT + i * 128, 128)])
            return 0

        lax.fori_loop(0, 5, zacc, 0)
        plsc.subcore_barrier()

        g_start(0, 0)
        g_start(1, 1)

        def outer(b2, _):
            b = 2 * b2
            g_wait(b, 0)
            do_scale(b, 0)
            s_sync(b, 0)
            g_start(b + 2, 0)
            g_wait(b + 1, 1)
            do_scale(b + 1, 1)
            s_sync(b + 1, 1)
            g_start(b + 3, 1)
            return 0

        lax.fori_loop(0, (nb - 2) // 2, outer, 0)
        b = nb - 2
        g_wait(b, 0)
        do_scale(b, 0)
        s_sync(b, 0)
        g_wait(b + 1, 1)
        do_scale(b + 1, 1)
        s_sync(b + 1, 1)
        plsc.subcore_barrier()
        pltpu.sync_copy(acc.at[pl.ds(s * _NPT, _NPT)],
                        out_hbm.at[c, m, pl.ds(s * _NPT, _NPT)])
        plsc.subcore_barrier()
        return 0

    lax.fori_loop(0, M, slice_loop, 0)


@functools.lru_cache(maxsize=None)
def _lhat_call(Mp, nb):
    return pl.kernel(
        _lhat_body,
        out_type=jax.ShapeDtypeStruct((_NC, Mp, _NPAD, _PW), jnp.float32),
        mesh=_sc_mesh(),
        name=f"lhat_m{Mp}",
        compiler_params=pltpu.CompilerParams(needs_layout_passes=False),
        scratch_types=[
            pltpu.VMEM((nb, _EB), jnp.int32),
            pltpu.VMEM((nb, _EB), jnp.float32),
            pltpu.VMEM((nb, _EB), jnp.int32),
            pltpu.VMEM((_EB, _PW), jnp.float32),
            pltpu.VMEM((_EB, _PW), jnp.float32),
            pltpu.VMEM_SHARED((_NPAD, _PW), jnp.float32),
            pltpu.SemaphoreType.DMA,
            pltpu.SemaphoreType.DMA,
        ],
    )


def _pairpad_body(p_ref, o_ref):
    o_ref[...] = (p_ref[0, 0] + p_ref[1, 0])[None]


def _pair_add_padded(p, bn=2000):
    _, Mp, _, C = p.shape
    grid_n = N_NODES // bn
    out = pl.pallas_call(
        _pairpad_body,
        grid=(Mp, grid_n),
        in_specs=[pl.BlockSpec((2, 1, bn, C), lambda m, i: (0, m, i, 0))],
        out_specs=pl.BlockSpec((1, bn, C), lambda m, i: (m, i, 0)),
        out_shape=jax.ShapeDtypeStruct((Mp, N_NODES, C), jnp.float32),
    )(p)
    return out.reshape(Mp * N_NODES, C)


def _sc_lhat(xf, src_all, dst3, lw3, Mp):
    nb = dst3.shape[1]
    part = _lhat_call(Mp, nb)(xf, src_all[:Mp], dst3, lw3)
    return _pair_add_padded(part)


def _prep_tconv_weights(w1, b1, w2, b2, w3, b3):
    def per_branch(w):
        m = jnp.transpose(w[:, :, 0, :], (2, 1, 0))
        return m.reshape(-1, m.shape[2])

    Wcat = jnp.concatenate([per_branch(w1), per_branch(w2), per_branch(w3)], axis=1)
    bcat = jnp.concatenate([b1, b2, b3]).reshape(1, -1)
    return Wcat, bcat


def kernel(x, edge_index, edge_weight,
           tc1_w1, tc1_b1, tc1_w2, tc1_b2, tc1_w3, tc1_b3,
           cheb_W, cheb_b,
           tc2_w1, tc2_b1, tc2_w2, tc2_b2, tc2_w3, tc2_b3,
           bn_gamma, bn_beta, lin_w, lin_b):
    src = edge_index[0].astype(jnp.int32)
    dst = edge_index[1].astype(jnp.int32)
    W1cat, b1cat = _prep_tconv_weights(tc1_w1, tc1_b1, tc1_w2, tc1_b2, tc1_w3, tc1_b3)
    W2cat, b2cat = _prep_tconv_weights(tc2_w1, tc2_b1, tc2_w2, tc2_b2, tc2_w3, tc2_b3)

    E = src.shape[0]
    nb = -(-E // (_NW * _EB))
    EP = _NW * _EB * nb
    pad = EP - E
    srcp = jnp.concatenate([src, jnp.zeros((pad,), jnp.int32)])
    dstp = jnp.concatenate([dst, jnp.zeros((pad,), jnp.int32)])
    ewp = jnp.concatenate([edge_weight, jnp.zeros((pad,), jnp.float32)])
    src3 = srcp.reshape(_NW, nb, _EB)
    dst3 = dstp.reshape(_NW, nb, _EB)
    ew3 = ewp.reshape(_NW, nb, _EB)
    Mpmax = (x.shape[1] - 2) // 2
    src_all = src3[None] + (jnp.arange(Mpmax, dtype=jnp.int32) * N_NODES)[:, None, None, None]

    ones_tab = jnp.ones((2 * N_NODES, _PW), jnp.float32)
    deg_srcall = jnp.stack([src3, src3 + N_NODES])
    part = _lhat_call(2, nb)(ones_tab, deg_srcall, src3, ew3)
    dis = _dis_from_partials(part[:, 0])
    lw3 = _sc_edge_norm(src3, dst3, ew3, dis)

    z = jnp.zeros((3, HID, HID), jnp.float32)
    Wbd = jnp.concatenate([
        jnp.concatenate([cheb_W, z], axis=2),
        jnp.concatenate([z, cheb_W], axis=2),
    ], axis=1)
    bbd = jnp.concatenate([cheb_b, cheb_b])

    h = x[0]
    for _ in range(3):
        t0 = _temporal_conv(h, W1cat, b1cat)
        M = t0.shape[0]
        Mp = M // 2
        t0p = (t0.reshape(Mp, 2, N_NODES, HID)
               .transpose(0, 2, 1, 3).reshape(Mp * N_NODES, _PW))
        y1 = _sc_lhat(t0p, src_all, dst3, lw3, Mp)
        y2 = _sc_lhat(y1, src_all, dst3, lw3, Mp)
        g = _cheb_combine(t0p, y1, y2, Wbd, bbd)
        gu = (g.reshape(Mp, N_NODES, 2, HID)
              .transpose(0, 2, 1, 3).reshape(M, N_NODES, HID))
        t2 = _temporal_conv(gu, W2cat, b2cat)
        h = _bn_relu(t2, bn_gamma, bn_beta)
    M2 = h.shape[0]
    out = _final_linear(h.reshape(M2 * N_NODES, F_IN), lin_w, lin_b)
    return out.reshape(1, M2, N_NODES, 1)


# speedup vs baseline: 13.9363x; 1.0000x over previous
import functools

import jax
import jax.numpy as jnp
from jax import lax
from jax.experimental import pallas as pl
from jax.experimental.pallas import tpu as pltpu
from jax.experimental.pallas import tpu_sc as plsc

N_NODES = 10000
F_IN = 128
HID = 64
KT = 3


def _tconv_body(x_ref, w_ref, b_ref, o_ref):
    T, BN, Cin = x_ref.shape
    Tout = T - 2
    C3 = w_ref.shape[1]
    Cout = C3 // 3
    xs = [x_ref[k:k + Tout].reshape(Tout * BN, Cin) for k in range(KT)]
    xcat = jnp.concatenate(xs, axis=1)
    acc = jnp.dot(xcat, w_ref[...], preferred_element_type=jnp.float32)
    acc = acc + b_ref[...]
    P = acc[:, :Cout]
    Q = acc[:, Cout:2 * Cout]
    R = acc[:, 2 * Cout:]
    H = jnp.maximum(P * jax.nn.sigmoid(Q) + R, 0.0)
    o_ref[...] = H.reshape(Tout, BN, Cout)


def _temporal_conv(x, wcat, bcat, bn=1000):
    T, N, Cin = x.shape
    C3 = wcat.shape[1]
    Cout = C3 // 3
    grid = N // bn
    return pl.pallas_call(
        _tconv_body,
        grid=(grid,),
        in_specs=[
            pl.BlockSpec((T, bn, Cin), lambda i: (0, i, 0)),
            pl.BlockSpec((KT * Cin, C3), lambda i: (0, 0)),
            pl.BlockSpec((1, C3), lambda i: (0, 0)),
        ],
        out_specs=pl.BlockSpec((T - 2, bn, Cout), lambda i: (0, i, 0)),
        out_shape=jax.ShapeDtypeStruct((T - 2, N, Cout), jnp.float32),
    )(x, wcat, bcat)


def _cheb_body(t0_ref, y1_ref, y2_ref, w_ref, b_ref, o_ref):
    t0 = t0_ref[...]
    a = jnp.dot(t0, w_ref[0], preferred_element_type=jnp.float32)
    a = a + jnp.dot(y1_ref[...], w_ref[1], preferred_element_type=jnp.float32)
    a = a + jnp.dot(2.0 * y2_ref[...] - t0, w_ref[2],
                    preferred_element_type=jnp.float32)
    o_ref[...] = jnp.maximum(a + b_ref[...], 0.0)


def _cheb_combine(t0f, y1, y2, cheb_W, cheb_b, br=2000):
    R, C = t0f.shape
    grid = R // br
    return pl.pallas_call(
        _cheb_body,
        grid=(grid,),
        in_specs=[
            pl.BlockSpec((br, C), lambda i: (i, 0)),
            pl.BlockSpec((br, C), lambda i: (i, 0)),
            pl.BlockSpec((br, C), lambda i: (i, 0)),
            pl.BlockSpec((3, C, C), lambda i: (0, 0, 0)),
            pl.BlockSpec((1, C), lambda i: (0, 0)),
        ],
        out_specs=pl.BlockSpec((br, C), lambda i: (i, 0)),
        out_shape=jax.ShapeDtypeStruct((R, C), jnp.float32),
    )(t0f, y1, y2, cheb_W, cheb_b.reshape(1, C))


def _bn_body(t_ref, g_ref, b_ref, o_ref):
    tb = t_ref[...]
    m = jnp.mean(tb, axis=(0, 2))
    ctr = tb - m[None, :, None]
    v = jnp.mean(ctr * ctr, axis=(0, 2))
    inv = 1.0 / jnp.sqrt(v + 1e-5)
    g = g_ref[0, 0]
    be = b_ref[0, 0]
    scale = (g * inv)[None, :, None]
    shift = (be - m * g * inv)[None, :, None]
    o_ref[...] = jnp.maximum(tb * scale + shift, 0.0)


def _bn_relu(t, gamma, beta, bn=1000):
    M2, N, C = t.shape
    grid = N // bn
    g2 = jnp.broadcast_to(gamma.reshape(grid, 1, bn), (grid, 8, bn))
    b2 = jnp.broadcast_to(beta.reshape(grid, 1, bn), (grid, 8, bn))
    return pl.pallas_call(
        _bn_body,
        grid=(grid,),
        in_specs=[
            pl.BlockSpec((M2, bn, C), lambda i: (0, i, 0)),
            pl.BlockSpec((1, 8, bn), lambda i: (i, 0, 0)),
            pl.BlockSpec((1, 8, bn), lambda i: (i, 0, 0)),
        ],
        out_specs=pl.BlockSpec((M2, bn, C), lambda i: (0, i, 0)),
        out_shape=jax.ShapeDtypeStruct((M2, N, C), jnp.float32),
    )(t, g2, b2)


def _lin_body(h_ref, w_ref, b_ref, o_ref):
    y = jnp.sum(h_ref[...] * w_ref[...], axis=1) + b_ref[0, 0]
    o_ref[...] = y[:, None]


def _final_linear(hf, lin_w, lin_b, br=4000):
    R, C = hf.shape
    grid = R // br
    return pl.pallas_call(
        _lin_body,
        grid=(grid,),
        in_specs=[
            pl.BlockSpec((br, C), lambda i: (i, 0)),
            pl.BlockSpec((1, C), lambda i: (0, 0)),
            pl.BlockSpec((1, 1), lambda i: (0, 0)),
        ],
        out_specs=pl.BlockSpec((br, 1), lambda i: (i, 0)),
        out_shape=jax.ShapeDtypeStruct((R, 1), jnp.float32),
    )(hf, lin_w, lin_b.reshape(1, 1))


_NC, _NS = 2, 16
_NW = _NC * _NS
_EB = 128
_NPAD = 10240
_NPT = _NPAD // _NS


def _sc_mesh():
    return plsc.VectorSubcoreMesh(core_axis_name="c", subcore_axis_name="s")


def _dis_body(p_ref, o_ref):
    deg = jnp.sum(p_ref[0] + p_ref[1], axis=1) * (1.0 / 128.0)
    safe = jnp.where(deg > 0, deg, 1.0)
    o_ref[...] = jnp.where(deg > 0, lax.rsqrt(safe), 0.0)[None, :]


def _dis_from_partials(part):
    out = pl.pallas_call(
        _dis_body,
        out_shape=jax.ShapeDtypeStruct((1, _NPAD), jnp.float32),
    )(part)
    return out.reshape(_NPAD)


def _lw_kernel():
    def body(src_hbm, dst_hbm, ew_hbm, dis_hbm, lw_hbm,
             srcv, dstv, ewv, disv, lwv):
        c = lax.axis_index("c")
        s = lax.axis_index("s")
        wid = c * _NS + s
        nb = srcv.shape[0]
        pltpu.sync_copy(src_hbm.at[wid], srcv)
        pltpu.sync_copy(dst_hbm.at[wid], dstv)
        pltpu.sync_copy(ew_hbm.at[wid], ewv)
        pltpu.sync_copy(dis_hbm, disv)

        def blk(b, _):
            def j16(j, _):
                sl = pl.ds(j * 16, 16)
                s16 = srcv[b, sl]
                d16 = dstv[b, sl]
                e16 = ewv[b, sl]
                dsc = plsc.load_gather(disv, [s16])
                ddc = plsc.load_gather(disv, [d16])
                lwv[b, sl] = (0.0 - dsc) * e16 * ddc
                return 0

            lax.fori_loop(0, _EB // 16, j16, 0)
            return 0

        lax.fori_loop(0, nb, blk, 0)
        pltpu.sync_copy(lwv, lw_hbm.at[wid])

    return body


def _sc_edge_norm(src3, dst3, ew3, dis):
    nb = src3.shape[1]
    return pl.kernel(
        _lw_kernel(),
        out_type=jax.ShapeDtypeStruct((_NW, nb, _EB), jnp.float32),
        mesh=_sc_mesh(),
        compiler_params=pltpu.CompilerParams(needs_layout_passes=False),
        scratch_types=[
            pltpu.VMEM((nb, _EB), jnp.int32),
            pltpu.VMEM((nb, _EB), jnp.int32),
            pltpu.VMEM((nb, _EB), jnp.float32),
            pltpu.VMEM((_NPAD,), jnp.float32),
            pltpu.VMEM((nb, _EB), jnp.float32),
        ],
    )(src3, dst3, ew3, dis)


_PW = 128


def _lhat_body(xf_hbm, srcall_hbm, dst_hbm, lw_hbm, out_hbm,
               dstv, lwv, srcmv, rows0, rows1, acc, gs0, gs1):
    c = lax.axis_index("c")
    s = lax.axis_index("s")
    wid = c * _NS + s
    nb = dstv.shape[0]
    M = srcall_hbm.shape[0]
    pltpu.sync_copy(dst_hbm.at[wid], dstv)
    pltpu.sync_copy(lw_hbm.at[wid], lwv)
    z16 = jnp.zeros((16,), jnp.float32)

    bufs = ((rows0, gs0), (rows1, gs1))

    def g_start(b, k):
        pltpu.async_copy(xf_hbm.at[srcmv.at[b]], bufs[k][0], bufs[k][1])

    def g_wait(b, k):
        pltpu.make_async_copy(xf_hbm.at[srcmv.at[b]], bufs[k][0],
                              bufs[k][1]).wait()

    def do_scale(b, k):
        rbuf = bufs[k][0]

        @plsc.parallel_loop(0, _EB, unroll=4)
        def scale(e):
            wv = plsc.load_gather(
                lwv, [jnp.full((16,), b, jnp.int32),
                      jnp.full((16,), e, jnp.int32)])
            for j in range(_PW // 16):
                sl = pl.ds(j * 16, 16)
                rbuf[e, sl] = rbuf[e, sl] * wv

    def s_sync(b, k):
        pltpu.sync_copy(bufs[k][0], acc.at[dstv.at[b]], add=True)

    def slice_loop(m, _):
        pltpu.sync_copy(srcall_hbm.at[m, wid], srcmv)

        def zrows(i, _):
            for j in range(_PW // 16):
                rows0[i, pl.ds(j * 16, 16)] = z16
            return 0

        lax.fori_loop(0, _EB, zrows, 0)

        def zacc(i, _):
            pltpu.sync_copy(rows0, acc.at[pl.ds(s * _NPT + i * 128, 128)])
            return 0

        lax.fori_loop(0, 5, zacc, 0)
        plsc.subcore_barrier()

        g_start(0, 0)
        g_start(1, 1)

        def outer(b2, _):
            b = 2 * b2
            g_wait(b, 0)
            do_scale(b, 0)
            s_sync(b, 0)
            g_start(b + 2, 0)
            g_wait(b + 1, 1)
            do_scale(b + 1, 1)
            s_sync(b + 1, 1)
            g_start(b + 3, 1)
            return 0

        lax.fori_loop(0, (nb - 2) // 2, outer, 0)
        b = nb - 2
        g_wait(b, 0)
        do_scale(b, 0)
        s_sync(b, 0)
        g_wait(b + 1, 1)
        do_scale(b + 1, 1)
        s_sync(b + 1, 1)
        plsc.subcore_barrier()
        pltpu.sync_copy(acc.at[pl.ds(s * _NPT, _NPT)],
                        out_hbm.at[c, m, pl.ds(s * _NPT, _NPT)])
        plsc.subcore_barrier()
        return 0

    lax.fori_loop(0, M, slice_loop, 0)


@functools.lru_cache(maxsize=None)
def _lhat_call(Mp, nb):
    return pl.kernel(
        _lhat_body,
        out_type=jax.ShapeDtypeStruct((_NC, Mp, _NPAD, _PW), jnp.float32),
        mesh=_sc_mesh(),
        name=f"lhat_m{Mp}",
        compiler_params=pltpu.CompilerParams(needs_layout_passes=False),
        scratch_types=[
            pltpu.VMEM((nb, _EB), jnp.int32),
            pltpu.VMEM((nb, _EB), jnp.float32),
            pltpu.VMEM((nb, _EB), jnp.int32),
            pltpu.VMEM((_EB, _PW), jnp.float32),
            pltpu.VMEM((_EB, _PW), jnp.float32),
            pltpu.VMEM_SHARED((_NPAD, _PW), jnp.float32),
            pltpu.SemaphoreType.DMA,
            pltpu.SemaphoreType.DMA,
        ],
    )


def _pairpad_body(p_ref, o_ref):
    o_ref[...] = (p_ref[0, 0] + p_ref[1, 0])[None]


def _pair_add_padded(p, bn=2000):
    _, Mp, _, C = p.shape
    grid_n = N_NODES // bn
    out = pl.pallas_call(
        _pairpad_body,
        grid=(Mp, grid_n),
        in_specs=[pl.BlockSpec((2, 1, bn, C), lambda m, i: (0, m, i, 0))],
        out_specs=pl.BlockSpec((1, bn, C), lambda m, i: (m, i, 0)),
        out_shape=jax.ShapeDtypeStruct((Mp, N_NODES, C), jnp.float32),
    )(p)
    return out.reshape(Mp * N_NODES, C)


def _sc_lhat(xf, src_all, dst3, lw3, Mp):
    nb = dst3.shape[1]
    part = _lhat_call(Mp, nb)(xf, src_all[:Mp], dst3, lw3)
    return _pair_add_padded(part)


def _prep_tconv_weights(w1, b1, w2, b2, w3, b3):
    def per_branch(w):
        m = jnp.transpose(w[:, :, 0, :], (2, 1, 0))
        return m.reshape(-1, m.shape[2])

    Wcat = jnp.concatenate([per_branch(w1), per_branch(w2), per_branch(w3)], axis=1)
    bcat = jnp.concatenate([b1, b2, b3]).reshape(1, -1)
    return Wcat, bcat


def kernel(x, edge_index, edge_weight,
           tc1_w1, tc1_b1, tc1_w2, tc1_b2, tc1_w3, tc1_b3,
           cheb_W, cheb_b,
           tc2_w1, tc2_b1, tc2_w2, tc2_b2, tc2_w3, tc2_b3,
           bn_gamma, bn_beta, lin_w, lin_b):
    src = edge_index[0].astype(jnp.int32)
    dst = edge_index[1].astype(jnp.int32)
    W1cat, b1cat = _prep_tconv_weights(tc1_w1, tc1_b1, tc1_w2, tc1_b2, tc1_w3, tc1_b3)
    W2cat, b2cat = _prep_tconv_weights(tc2_w1, tc2_b1, tc2_w2, tc2_b2, tc2_w3, tc2_b3)

    E = src.shape[0]
    nb = -(-E // (_NW * _EB))
    EP = _NW * _EB * nb
    pad = EP - E
    srcp = jnp.concatenate([src, jnp.zeros((pad,), jnp.int32)])
    dstp = jnp.concatenate([dst, jnp.zeros((pad,), jnp.int32)])
    ewp = jnp.concatenate([edge_weight, jnp.zeros((pad,), jnp.float32)])
    src3 = srcp.reshape(_NW, nb, _EB)
    dst3 = dstp.reshape(_NW, nb, _EB)
    ew3 = ewp.reshape(_NW, nb, _EB)
    Mpmax = (x.shape[1] - 2) // 2
    src_all = src3[None] + (jnp.arange(Mpmax, dtype=jnp.int32) * N_NODES)[:, None, None, None]

    ones_tab = jnp.ones((2 * N_NODES, _PW), jnp.float32)
    deg_srcall = jnp.stack([src3, src3 + N_NODES])
    part = _lhat_call(2, nb)(ones_tab, deg_srcall, src3, ew3)
    dis = _dis_from_partials(part[:, 0])
    lw3 = _sc_edge_norm(src3, dst3, ew3, dis)

    z = jnp.zeros((3, HID, HID), jnp.float32)
    Wbd = jnp.concatenate([
        jnp.concatenate([cheb_W, z], axis=2),
        jnp.concatenate([z, cheb_W], axis=2),
    ], axis=1)
    bbd = jnp.concatenate([cheb_b, cheb_b])

    h = x[0]
    for _ in range(3):
        t0 = _temporal_conv(h, W1cat, b1cat)
        M = t0.shape[0]
        Mp = M // 2
        t0p = (t0.reshape(Mp, 2, N_NODES, HID)
               .transpose(0, 2, 1, 3).reshape(Mp * N_NODES, _PW))
        y1 = _sc_lhat(t0p, src_all, dst3, lw3, Mp)
        y2 = _sc_lhat(y1, src_all, dst3, lw3, Mp)
        g = _cheb_combine(t0p, y1, y2, Wbd, bbd)
        gu = (g.reshape(Mp, N_NODES, 2, HID)
              .transpose(0, 2, 1, 3).reshape(M, N_NODES, HID))
        t2 = _temporal_conv(gu, W2cat, b2cat)
        h = _bn_relu(t2, bn_gamma, bn_beta)
    M2 = h.shape[0]
    out = _final_linear(h.reshape(M2 * N_NODES, F_IN), lin_w, lin_b)
    return out.reshape(1, M2, N_NODES, 1)
